# baseline (speedup 1.0000x reference)
import functools

import jax
import jax.numpy as jnp
from jax import lax
from jax.experimental import pallas as pl
from jax.experimental.pallas import tpu as pltpu, tpu_sc as plsc

_NC, _NS = 2, 16
_NW = _NC * _NS
_CCHUNK = 4


def _sc_copy_body(in_hbm, out_hbm, buf0, buf1,
                  isem0, isem1, osem0, osem1):
    cid = lax.axis_index("c")
    sid = lax.axis_index("s")
    wid = sid * _NC + cid
    b = in_hbm.shape[0]
    ch_per_w = in_hbm.shape[1] // _NW
    chunks_per_b = ch_per_w // _CCHUNK
    n = b * chunks_per_b
    bufs = (buf0, buf1)
    isems = (isem0, isem1)
    osems = (osem0, osem1)

    def region(ref, i):
        bi, j = divmod(i, chunks_per_b)
        c0 = wid * ch_per_w + j * _CCHUNK
        return ref.at[pl.ds(bi, 1), pl.ds(c0, _CCHUNK)]

    copies_in = [None, None]
    copies_out = [None, None]
    copies_in[0] = pltpu.async_copy(region(in_hbm, 0), bufs[0], isems[0])
    for i in range(n):
        s = i % 2
        o = (i + 1) % 2
        if i + 1 < n:
            if i >= 1:
                copies_out[o].wait()
            copies_in[o] = pltpu.async_copy(region(in_hbm, i + 1),
                                            bufs[o], isems[o])
        copies_in[s].wait()
        copies_out[s] = pltpu.async_copy(bufs[s], region(out_hbm, i),
                                         osems[s])
    copies_out[(n - 1) % 2].wait()
    if n >= 2:
        copies_out[n % 2].wait()


def kernel(input):
    b, c, h, w = input.shape
    mesh = plsc.VectorSubcoreMesh(core_axis_name="c", subcore_axis_name="s",
                                  num_cores=_NC, num_subcores=_NS)
    run = pl.kernel(
        _sc_copy_body,
        out_type=jax.ShapeDtypeStruct(input.shape, input.dtype),
        mesh=mesh,
        scratch_types=[
            pltpu.VMEM((1, _CCHUNK, h, w), input.dtype),
            pltpu.VMEM((1, _CCHUNK, h, w), input.dtype),
            pltpu.SemaphoreType.DMA,
            pltpu.SemaphoreType.DMA,
            pltpu.SemaphoreType.DMA,
            pltpu.SemaphoreType.DMA,
        ],
        compiler_params=pltpu.CompilerParams(use_tc_tiling_on_sc=True),
    )
    return run(input)


# speedup vs baseline: 4.3155x; 4.3155x over previous
import jax
import jax.numpy as jnp
from jax.experimental import pallas as pl
from jax.experimental.pallas import tpu as pltpu

_BLOCK_ROWS = 2048


def _copy_body(in_ref, out_ref):
    out_ref[...] = in_ref[...]


def kernel(input):
    b, c, h, w = input.shape
    flat = jnp.transpose(input, (0, 2, 3, 1)).reshape(b * h * w, c)
    rows = flat.shape[0]
    out = pl.pallas_call(
        _copy_body,
        out_shape=jax.ShapeDtypeStruct(flat.shape, flat.dtype),
        grid=(rows // _BLOCK_ROWS,),
        in_specs=[pl.BlockSpec((_BLOCK_ROWS, c), lambda i: (i, 0))],
        out_specs=pl.BlockSpec((_BLOCK_ROWS, c), lambda i: (i, 0)),
    )(flat)
    return jnp.transpose(out.reshape(b, h, w, c), (0, 3, 1, 2))


# speedup vs baseline: 4.8030x; 1.1130x over previous
import jax
import jax.numpy as jnp
from jax.experimental import pallas as pl
from jax.experimental.pallas import tpu as pltpu

_BLOCK_ROWS = 7168


def _copy_body(in_ref, out_ref):
    out_ref[...] = in_ref[...]


def kernel(input):
    b, c, h, w = input.shape
    flat = jnp.transpose(input, (0, 2, 3, 1)).reshape(b * h * w, c)
    rows = flat.shape[0]
    out = pl.pallas_call(
        _copy_body,
        out_shape=jax.ShapeDtypeStruct(flat.shape, flat.dtype),
        grid=(rows // _BLOCK_ROWS,),
        in_specs=[pl.BlockSpec((_BLOCK_ROWS, c), lambda i: (i, 0))],
        out_specs=pl.BlockSpec((_BLOCK_ROWS, c), lambda i: (i, 0)),
    )(flat)
    return jnp.transpose(out.reshape(b, h, w, c), (0, 3, 1, 2))


# speedup vs baseline: 4.8308x; 1.0058x over previous
import jax
import jax.numpy as jnp
from jax.experimental import pallas as pl
from jax.experimental.pallas import tpu as pltpu

_BLOCK_ROWS = 12544


def _copy_body(in_ref, out_ref):
    out_ref[...] = in_ref[...]


def kernel(input):
    b, c, h, w = input.shape
    flat = jnp.transpose(input, (0, 2, 3, 1)).reshape(b * h * w, c)
    rows = flat.shape[0]
    out = pl.pallas_call(
        _copy_body,
        out_shape=jax.ShapeDtypeStruct(flat.shape, flat.dtype),
        grid=(rows // _BLOCK_ROWS,),
        in_specs=[pl.BlockSpec((_BLOCK_ROWS, c), lambda i: (i, 0))],
        out_specs=pl.BlockSpec((_BLOCK_ROWS, c), lambda i: (i, 0)),
    )(flat)
    return jnp.transpose(out.reshape(b, h, w, c), (0, 3, 1, 2))
